# baseline (speedup 1.0000x reference)
import functools
import math

import jax
import jax.numpy as jnp
from jax import lax
from jax.experimental import pallas as pl
from jax.experimental.pallas import tpu as pltpu

_WINDOW = 4
_K = 8
_QBLK = 256
_CBLK = 256


def _fold_kernel(wv_ref, wm2_ref, wo_ref, bv_ref, bm2_ref, bo_ref,
                 wvpT_ref, wlT_ref, bvp_ref, ball_ref):
    D = wv_ref.shape[0]
    wo = wo_ref[...]
    wol = wo[:, :D]
    wog = wo[:, D:]
    wv = wv_ref[...]
    wm2 = wm2_ref[...]
    wvpT_ref[...] = lax.dot_general(
        wv, wog, (((0,), (1,)), ((), ())),
        preferred_element_type=jnp.float32).astype(jnp.bfloat16)
    wlT_ref[...] = lax.dot_general(
        wm2, wol, (((0,), (1,)), ((), ())),
        preferred_element_type=jnp.float32).astype(jnp.bfloat16)
    zD = jnp.zeros((1, D), jnp.float32)
    lhs = jnp.concatenate([
        jnp.concatenate([bm2_ref[...], zD], axis=1),
        jnp.concatenate([zD, bv_ref[...]], axis=1),
    ], axis=0)
    y = lax.dot_general(lhs, wo, (((1,), (1,)), ((), ())),
                        preferred_element_type=jnp.float32)
    ball_ref[...] = bo_ref[...] + y[0:1]
    bvp_ref[...] = y[1:2]


def _proj_kernel(nblk, x_ref, wk_ref, wvpT_ref, wm1_ref, wlT_ref,
                 bk_ref, bvp_ref, bm1_ref, ball_ref,
                 k_ref, vp_ref, loc_ref, tail_ref):
    i = pl.program_id(0)
    qi = lax.rem(i, nblk)
    D = x_ref.shape[1]
    QB = x_ref.shape[0]
    x = x_ref[...]
    k_ref[...] = lax.dot_general(
        x, wk_ref[...], (((1,), (1,)), ((), ())),
        preferred_element_type=jnp.float32) + bk_ref[...]
    xb = x.astype(jnp.bfloat16)
    vp = lax.dot_general(
        xb, wvpT_ref[...], (((1,), (0,)), ((), ())),
        preferred_element_type=jnp.float32) + bvp_ref[...]
    vp_ref[...] = vp.astype(jnp.bfloat16)
    wm1 = wm1_ref[...]
    a = lax.dot_general(
        xb, wm1[:, :D], (((1,), (1,)), ((), ())),
        preferred_element_type=jnp.float32) + bm1_ref[...]
    bn = lax.dot_general(
        xb, wm1[:, D:], (((1,), (1,)), ((), ())),
        preferred_element_type=jnp.float32)
    prev = jnp.where(qi == 0, 0.0, tail_ref[...])
    acc = jnp.zeros_like(a)
    for w in range(1, _WINDOW + 1):
        sh = jnp.concatenate([prev[8 - w:, :], bn[:QB - w, :]], axis=0)
        z = a + sh
        acc = acc + z * jax.nn.sigmoid(z)
    tail_ref[...] = bn[QB - 8:, :]
    pre = (acc * (1.0 / _WINDOW)).astype(jnp.bfloat16)
    loc_ref[...] = lax.dot_general(
        pre, wlT_ref[...], (((1,), (0,)), ((), ())),
        preferred_element_type=jnp.float32) + ball_ref[...]


def _attn_kernel(x_ref, wq_ref, bq_ref, k_ref, vp_ref, loc_ref, o_ref,
                 s_ref):
    qi = pl.program_id(1)
    QB = x_ref.shape[1]
    N = k_ref.shape[1]
    D = x_ref.shape[2]
    nch = N // _CBLK
    scale = 1.0 / math.sqrt(D)
    neg = jnp.float32(-jnp.inf)
    f32 = jnp.float32
    q = lax.dot_general(
        x_ref[0], wq_ref[...], (((1,), (1,)), ((), ())),
        preferred_element_type=f32) + bq_ref[...]

    rloc = lax.broadcasted_iota(jnp.int32, (QB, _CBLK), 0)
    cloc = lax.broadcasted_iota(jnp.int32, (QB, _CBLK), 1)

    def scores_body(ki, _):
        kblk = k_ref[0, pl.ds(ki * _CBLK, _CBLK), :]
        sb = lax.dot_general(
            q, kblk, (((1,), (1,)), ((), ())),
            preferred_element_type=f32) * scale
        sb = jnp.where(ki * _CBLK + cloc <= qi * QB + rloc, sb, neg)
        s_ref[:, pl.ds(ki * _CBLK, _CBLK)] = sb
        return 0

    def blank_body(ki, _):
        s_ref[:, pl.ds(ki * _CBLK, _CBLK)] = jnp.full((QB, _CBLK), neg, f32)
        return 0

    lax.fori_loop(0, qi + 1, scores_body, 0)
    lax.fori_loop(qi + 1, nch, blank_body, 0)

    s = s_ref[...]
    m0 = jnp.max(s, axis=1, keepdims=True)
    mprev = m0
    for _ in range(_K - 1):
        mprev = jnp.max(jnp.where(s < mprev, s, neg), axis=1, keepdims=True)
    t = mprev
    cnt = jnp.sum(((s >= t) & (s > neg)).astype(jnp.int32), axis=1,
                  keepdims=True)
    rowg = qi * QB + lax.broadcasted_iota(jnp.int32, (QB, 1), 0)
    ok = jnp.all(cnt == jnp.minimum(rowg + 1, _K))

    def fast(_):
        def body(ki, carry):
            g, z = carry
            sb = s_ref[:, pl.ds(ki * _CBLK, _CBLK)]
            pb = jnp.where(sb >= t, jnp.exp(sb - m0), 0.0)
            z = z + jnp.sum(pb, axis=1, keepdims=True)
            vb = vp_ref[0, pl.ds(ki * _CBLK, _CBLK), :]
            g = g + lax.dot_general(
                pb.astype(jnp.bfloat16), vb, (((1,), (0,)), ((), ())),
                preferred_element_type=f32)
            return (g, z)
        return lax.fori_loop(
            0, qi + 1, body,
            (jnp.zeros((QB, D), f32), jnp.zeros((QB, 1), f32)))

    def slow(_):
        col = lax.broadcasted_iota(jnp.int32, (QB, N), 1)
        work = s
        for _ in range(_K):
            m = jnp.max(work, axis=1, keepdims=True)
            am = jnp.min(jnp.where(work == m, col, N), axis=1, keepdims=True)
            work = jnp.where(col == am, neg, work)
        mask = work != s
        p = jnp.where(mask, jnp.exp(s - m0), 0.0)
        z = jnp.sum(p, axis=1, keepdims=True)
        g = lax.dot_general(
            p.astype(jnp.bfloat16), vp_ref[0], (((1,), (0,)), ((), ())),
            preferred_element_type=f32)
        return (g, z)

    g, z = lax.cond(ok, fast, slow, 0)
    o_ref[0] = g / z + loc_ref[0]


def kernel(mu, Wq, bq, Wk, bk, Wv, bv, Wm1, bm1, Wm2, bm2, Wo, bo):
    B, N, D = mu.shape
    f32 = jnp.float32
    bf16 = jnp.bfloat16

    wvpT, wlT, bvp, ball = pl.pallas_call(
        _fold_kernel,
        out_shape=[
            jax.ShapeDtypeStruct((D, D), bf16),
            jax.ShapeDtypeStruct((D, D), bf16),
            jax.ShapeDtypeStruct((1, D), f32),
            jax.ShapeDtypeStruct((1, D), f32),
        ],
    )(Wv, Wm2, Wo, bv[None, :], bm2[None, :], bo[None, :])

    mu_flat = mu.reshape(B * N, D)
    nblk = N // _QBLK
    kk, vp, loc = pl.pallas_call(
        functools.partial(_proj_kernel, nblk),
        grid=(B * nblk,),
        in_specs=[
            pl.BlockSpec((_QBLK, D), lambda i: (i, 0)),
            pl.BlockSpec((D, D), lambda i: (0, 0)),
            pl.BlockSpec((D, D), lambda i: (0, 0)),
            pl.BlockSpec((D, 2 * D), lambda i: (0, 0)),
            pl.BlockSpec((D, D), lambda i: (0, 0)),
            pl.BlockSpec((1, D), lambda i: (0, 0)),
            pl.BlockSpec((1, D), lambda i: (0, 0)),
            pl.BlockSpec((1, D), lambda i: (0, 0)),
            pl.BlockSpec((1, D), lambda i: (0, 0)),
        ],
        out_specs=[
            pl.BlockSpec((_QBLK, D), lambda i: (i, 0)),
            pl.BlockSpec((_QBLK, D), lambda i: (i, 0)),
            pl.BlockSpec((_QBLK, D), lambda i: (i, 0)),
        ],
        out_shape=[
            jax.ShapeDtypeStruct((B * N, D), f32),
            jax.ShapeDtypeStruct((B * N, D), bf16),
            jax.ShapeDtypeStruct((B * N, D), f32),
        ],
        scratch_shapes=[pltpu.VMEM((8, D), f32)],
    )(mu_flat, Wk, wvpT, Wm1.astype(bf16), wlT,
      bk[None, :], bvp, bm1[None, :], ball)

    kk = kk.reshape(B, N, D)
    vp = vp.reshape(B, N, D)
    loc = loc.reshape(B, N, D)

    out = pl.pallas_call(
        _attn_kernel,
        grid=(B, nblk),
        in_specs=[
            pl.BlockSpec((1, _QBLK, D), lambda b, i: (b, i, 0)),
            pl.BlockSpec((D, D), lambda b, i: (0, 0)),
            pl.BlockSpec((1, D), lambda b, i: (0, 0)),
            pl.BlockSpec((1, N, D), lambda b, i: (b, 0, 0)),
            pl.BlockSpec((1, N, D), lambda b, i: (b, 0, 0)),
            pl.BlockSpec((1, _QBLK, D), lambda b, i: (b, i, 0)),
        ],
        out_specs=pl.BlockSpec((1, _QBLK, D), lambda b, i: (b, i, 0)),
        out_shape=jax.ShapeDtypeStruct((B, N, D), f32),
        scratch_shapes=[pltpu.VMEM((_QBLK, N), f32)],
    )(mu, Wq, bq[None, :], kk, vp, loc)
    return out


# speedup vs baseline: 1.0001x; 1.0001x over previous
import functools
import math

import jax
import jax.numpy as jnp
from jax import lax
from jax.experimental import pallas as pl
from jax.experimental.pallas import tpu as pltpu

_WINDOW = 4
_K = 8
_QBLK = 256
_CBLK = 256


def _fold_kernel(wv_ref, wm2_ref, wo_ref, bv_ref, bm2_ref, bo_ref,
                 wvpT_ref, wlT_ref, bvp_ref, ball_ref):
    D = wv_ref.shape[0]
    wo = wo_ref[...]
    wol = wo[:, :D]
    wog = wo[:, D:]
    wv = wv_ref[...]
    wm2 = wm2_ref[...]
    wvpT_ref[...] = lax.dot_general(
        wv, wog, (((0,), (1,)), ((), ())),
        preferred_element_type=jnp.float32).astype(jnp.bfloat16)
    wlT_ref[...] = lax.dot_general(
        wm2, wol, (((0,), (1,)), ((), ())),
        preferred_element_type=jnp.float32).astype(jnp.bfloat16)
    zD = jnp.zeros((1, D), jnp.float32)
    lhs = jnp.concatenate([
        jnp.concatenate([bm2_ref[...], zD], axis=1),
        jnp.concatenate([zD, bv_ref[...]], axis=1),
    ], axis=0)
    y = lax.dot_general(lhs, wo, (((1,), (1,)), ((), ())),
                        preferred_element_type=jnp.float32)
    ball_ref[...] = bo_ref[...] + y[0:1]
    bvp_ref[...] = y[1:2]


def _proj_kernel(x_ref, wk_ref, wvpT_ref, wm1_ref, wlT_ref,
                 bk_ref, bvp_ref, bm1_ref, ball_ref,
                 k_ref, vp_ref, loc_ref, tail_ref):
    qi = pl.program_id(1)
    D = x_ref.shape[2]
    QB = x_ref.shape[1]
    x = x_ref[0]
    k_ref[0] = lax.dot_general(
        x, wk_ref[...], (((1,), (1,)), ((), ())),
        preferred_element_type=jnp.float32) + bk_ref[...]
    xb = x.astype(jnp.bfloat16)
    vp = lax.dot_general(
        xb, wvpT_ref[...], (((1,), (0,)), ((), ())),
        preferred_element_type=jnp.float32) + bvp_ref[...]
    vp_ref[0] = vp.astype(jnp.bfloat16)
    wm1 = wm1_ref[...]
    a = lax.dot_general(
        xb, wm1[:, :D], (((1,), (1,)), ((), ())),
        preferred_element_type=jnp.float32) + bm1_ref[...]
    bn = lax.dot_general(
        xb, wm1[:, D:], (((1,), (1,)), ((), ())),
        preferred_element_type=jnp.float32)
    prev = jnp.where(qi == 0, 0.0, tail_ref[...])
    acc = jnp.zeros_like(a)
    for w in range(1, _WINDOW + 1):
        sh = jnp.concatenate([prev[8 - w:, :], bn[:QB - w, :]], axis=0)
        z = a + sh
        acc = acc + z * jax.nn.sigmoid(z)
    tail_ref[...] = bn[QB - 8:, :]
    pre = (acc * (1.0 / _WINDOW)).astype(jnp.bfloat16)
    loc_ref[0] = lax.dot_general(
        pre, wlT_ref[...], (((1,), (0,)), ((), ())),
        preferred_element_type=jnp.float32) + ball_ref[...]


def _attn_kernel(x_ref, wq_ref, bq_ref, k_ref, vp_ref, loc_ref, o_ref,
                 s_ref):
    qi = pl.program_id(1)
    QB = x_ref.shape[1]
    N = k_ref.shape[1]
    D = x_ref.shape[2]
    nch = N // _CBLK
    scale = 1.0 / math.sqrt(D)
    neg = jnp.float32(-jnp.inf)
    f32 = jnp.float32
    q = lax.dot_general(
        x_ref[0], wq_ref[...], (((1,), (1,)), ((), ())),
        preferred_element_type=f32) + bq_ref[...]

    rloc = lax.broadcasted_iota(jnp.int32, (QB, _CBLK), 0)
    cloc = lax.broadcasted_iota(jnp.int32, (QB, _CBLK), 1)

    def scores_body(ki, _):
        kblk = k_ref[0, pl.ds(ki * _CBLK, _CBLK), :]
        sb = lax.dot_general(
            q, kblk, (((1,), (1,)), ((), ())),
            preferred_element_type=f32) * scale
        sb = jnp.where(ki * _CBLK + cloc <= qi * QB + rloc, sb, neg)
        s_ref[:, pl.ds(ki * _CBLK, _CBLK)] = sb
        return 0

    def blank_body(ki, _):
        s_ref[:, pl.ds(ki * _CBLK, _CBLK)] = jnp.full((QB, _CBLK), neg, f32)
        return 0

    lax.fori_loop(0, qi + 1, scores_body, 0)
    lax.fori_loop(qi + 1, nch, blank_body, 0)

    s = s_ref[...]
    m0 = jnp.max(s, axis=1, keepdims=True)
    mprev = m0
    for _ in range(_K - 1):
        mprev = jnp.max(jnp.where(s < mprev, s, neg), axis=1, keepdims=True)
    t = mprev
    cnt = jnp.sum(((s >= t) & (s > neg)).astype(jnp.int32), axis=1,
                  keepdims=True)
    rowg = qi * QB + lax.broadcasted_iota(jnp.int32, (QB, 1), 0)
    ok = jnp.all(cnt == jnp.minimum(rowg + 1, _K))

    def fast(_):
        def body(ki, carry):
            g, z = carry
            sb = s_ref[:, pl.ds(ki * _CBLK, _CBLK)]
            pb = jnp.where(sb >= t, jnp.exp(sb - m0), 0.0)
            z = z + jnp.sum(pb, axis=1, keepdims=True)
            vb = vp_ref[0, pl.ds(ki * _CBLK, _CBLK), :]
            g = g + lax.dot_general(
                pb.astype(jnp.bfloat16), vb, (((1,), (0,)), ((), ())),
                preferred_element_type=f32)
            return (g, z)
        return lax.fori_loop(
            0, qi + 1, body,
            (jnp.zeros((QB, D), f32), jnp.zeros((QB, 1), f32)))

    def slow(_):
        col = lax.broadcasted_iota(jnp.int32, (QB, N), 1)
        work = s
        for _ in range(_K):
            m = jnp.max(work, axis=1, keepdims=True)
            am = jnp.min(jnp.where(work == m, col, N), axis=1, keepdims=True)
            work = jnp.where(col == am, neg, work)
        mask = work != s
        p = jnp.where(mask, jnp.exp(s - m0), 0.0)
        z = jnp.sum(p, axis=1, keepdims=True)
        g = lax.dot_general(
            p.astype(jnp.bfloat16), vp_ref[0], (((1,), (0,)), ((), ())),
            preferred_element_type=f32)
        return (g, z)

    g, z = lax.cond(ok, fast, slow, 0)
    o_ref[0] = g / z + loc_ref[0]


def kernel(mu, Wq, bq, Wk, bk, Wv, bv, Wm1, bm1, Wm2, bm2, Wo, bo):
    B, N, D = mu.shape
    f32 = jnp.float32
    bf16 = jnp.bfloat16

    wvpT, wlT, bvp, ball = pl.pallas_call(
        _fold_kernel,
        out_shape=[
            jax.ShapeDtypeStruct((D, D), bf16),
            jax.ShapeDtypeStruct((D, D), bf16),
            jax.ShapeDtypeStruct((1, D), f32),
            jax.ShapeDtypeStruct((1, D), f32),
        ],
    )(Wv, Wm2, Wo, bv[None, :], bm2[None, :], bo[None, :])

    nblk = N // _QBLK
    kk, vp, loc = pl.pallas_call(
        _proj_kernel,
        grid=(B, nblk),
        in_specs=[
            pl.BlockSpec((1, _QBLK, D), lambda b, i: (b, i, 0)),
            pl.BlockSpec((D, D), lambda b, i: (0, 0)),
            pl.BlockSpec((D, D), lambda b, i: (0, 0)),
            pl.BlockSpec((D, 2 * D), lambda b, i: (0, 0)),
            pl.BlockSpec((D, D), lambda b, i: (0, 0)),
            pl.BlockSpec((1, D), lambda b, i: (0, 0)),
            pl.BlockSpec((1, D), lambda b, i: (0, 0)),
            pl.BlockSpec((1, D), lambda b, i: (0, 0)),
            pl.BlockSpec((1, D), lambda b, i: (0, 0)),
        ],
        out_specs=[
            pl.BlockSpec((1, _QBLK, D), lambda b, i: (b, i, 0)),
            pl.BlockSpec((1, _QBLK, D), lambda b, i: (b, i, 0)),
            pl.BlockSpec((1, _QBLK, D), lambda b, i: (b, i, 0)),
        ],
        out_shape=[
            jax.ShapeDtypeStruct((B, N, D), f32),
            jax.ShapeDtypeStruct((B, N, D), bf16),
            jax.ShapeDtypeStruct((B, N, D), f32),
        ],
        scratch_shapes=[pltpu.VMEM((8, D), f32)],
        compiler_params=pltpu.CompilerParams(
            dimension_semantics=("parallel", "arbitrary")),
    )(mu, Wk, wvpT, Wm1.astype(bf16), wlT,
      bk[None, :], bvp, bm1[None, :], ball)

    out = pl.pallas_call(
        _attn_kernel,
        grid=(B, nblk),
        in_specs=[
            pl.BlockSpec((1, _QBLK, D), lambda b, i: (b, i, 0)),
            pl.BlockSpec((D, D), lambda b, i: (0, 0)),
            pl.BlockSpec((1, D), lambda b, i: (0, 0)),
            pl.BlockSpec((1, N, D), lambda b, i: (b, 0, 0)),
            pl.BlockSpec((1, N, D), lambda b, i: (b, 0, 0)),
            pl.BlockSpec((1, _QBLK, D), lambda b, i: (b, i, 0)),
        ],
        out_specs=pl.BlockSpec((1, _QBLK, D), lambda b, i: (b, i, 0)),
        out_shape=jax.ShapeDtypeStruct((B, N, D), f32),
        scratch_shapes=[pltpu.VMEM((_QBLK, N), f32)],
        compiler_params=pltpu.CompilerParams(
            dimension_semantics=("parallel", "parallel")),
    )(mu, Wq, bq[None, :], kk, vp, loc)
    return out


# speedup vs baseline: 1.0184x; 1.0183x over previous
import functools
import math

import jax
import jax.numpy as jnp
from jax import lax
from jax.experimental import pallas as pl
from jax.experimental.pallas import tpu as pltpu

_WINDOW = 4
_K = 8
_QBLK = 256
_CBLK = 256


def _fold_kernel(wv_ref, wm2_ref, wo_ref, bv_ref, bm2_ref, bo_ref,
                 wvpT_ref, wlT_ref, bvp_ref, ball_ref):
    D = wv_ref.shape[0]
    wo = wo_ref[...]
    wol = wo[:, :D]
    wog = wo[:, D:]
    wv = wv_ref[...]
    wm2 = wm2_ref[...]
    wvpT_ref[...] = lax.dot_general(
        wv, wog, (((0,), (1,)), ((), ())),
        preferred_element_type=jnp.float32).astype(jnp.bfloat16)
    wlT_ref[...] = lax.dot_general(
        wm2, wol, (((0,), (1,)), ((), ())),
        preferred_element_type=jnp.float32).astype(jnp.bfloat16)
    zD = jnp.zeros((1, D), jnp.float32)
    lhs = jnp.concatenate([
        jnp.concatenate([bm2_ref[...], zD], axis=1),
        jnp.concatenate([zD, bv_ref[...]], axis=1),
    ], axis=0)
    y = lax.dot_general(lhs, wo, (((1,), (1,)), ((), ())),
                        preferred_element_type=jnp.float32)
    ball_ref[...] = bo_ref[...] + y[0:1]
    bvp_ref[...] = y[1:2]


def _proj_kernel(x_ref, wk_ref, wvpT_ref, wm1_ref, wlT_ref,
                 bk_ref, bvp_ref, bm1_ref, ball_ref,
                 k_ref, vp_ref, loc_ref, tail_ref):
    qi = pl.program_id(1)
    D = x_ref.shape[2]
    QB = x_ref.shape[1]
    x = x_ref[0]
    k_ref[0] = lax.dot_general(
        x, wk_ref[...], (((1,), (1,)), ((), ())),
        preferred_element_type=jnp.float32) + bk_ref[...]
    xb = x.astype(jnp.bfloat16)
    vp = lax.dot_general(
        xb, wvpT_ref[...], (((1,), (0,)), ((), ())),
        preferred_element_type=jnp.float32) + bvp_ref[...]
    vp_ref[0] = vp.astype(jnp.bfloat16)
    wm1 = wm1_ref[...]
    a = (lax.dot_general(
        xb, wm1[:, :D], (((1,), (1,)), ((), ())),
        preferred_element_type=jnp.float32) + bm1_ref[...]).astype(jnp.bfloat16)
    bn = lax.dot_general(
        xb, wm1[:, D:], (((1,), (1,)), ((), ())),
        preferred_element_type=jnp.float32).astype(jnp.bfloat16)
    prev = jnp.where(qi == 0, 0.0, tail_ref[...]).astype(jnp.bfloat16)
    acc = jnp.zeros(a.shape, jnp.float32)
    for w in range(1, _WINDOW + 1):
        sh = jnp.concatenate([prev[8 - w:, :], bn[:QB - w, :]], axis=0)
        z = a + sh
        acc = acc + (z * jax.nn.sigmoid(z)).astype(jnp.float32)
    tail_ref[...] = bn[QB - 8:, :].astype(jnp.float32)
    pre = (acc * (1.0 / _WINDOW)).astype(jnp.bfloat16)
    loc_ref[0] = lax.dot_general(
        pre, wlT_ref[...], (((1,), (0,)), ((), ())),
        preferred_element_type=jnp.float32) + ball_ref[...]


def _attn_kernel(x_ref, wq_ref, bq_ref, k_ref, vp_ref, loc_ref, o_ref,
                 s_ref):
    qi = pl.program_id(1)
    QB = x_ref.shape[1]
    N = k_ref.shape[1]
    D = x_ref.shape[2]
    nch = N // _CBLK
    scale = 1.0 / math.sqrt(D)
    neg = jnp.float32(-jnp.inf)
    f32 = jnp.float32
    q = lax.dot_general(
        x_ref[0], wq_ref[...], (((1,), (1,)), ((), ())),
        preferred_element_type=f32) + bq_ref[...]

    rloc = lax.broadcasted_iota(jnp.int32, (QB, _CBLK), 0)
    cloc = lax.broadcasted_iota(jnp.int32, (QB, _CBLK), 1)

    def scores_body(ki, _):
        kblk = k_ref[0, pl.ds(ki * _CBLK, _CBLK), :]
        sb = lax.dot_general(
            q, kblk, (((1,), (1,)), ((), ())),
            preferred_element_type=f32) * scale
        sb = jnp.where(ki * _CBLK + cloc <= qi * QB + rloc, sb, neg)
        s_ref[:, pl.ds(ki * _CBLK, _CBLK)] = sb
        return 0

    def blank_body(ki, _):
        s_ref[:, pl.ds(ki * _CBLK, _CBLK)] = jnp.full((QB, _CBLK), neg, f32)
        return 0

    lax.fori_loop(0, qi + 1, scores_body, 0)
    lax.fori_loop(qi + 1, nch, blank_body, 0)

    s = s_ref[...]
    m0 = jnp.max(s, axis=1, keepdims=True)
    mprev = m0
    for _ in range(_K - 1):
        mprev = jnp.max(jnp.where(s < mprev, s, neg), axis=1, keepdims=True)
    t = mprev
    cnt = jnp.sum(((s >= t) & (s > neg)).astype(jnp.int32), axis=1,
                  keepdims=True)
    rowg = qi * QB + lax.broadcasted_iota(jnp.int32, (QB, 1), 0)
    ok = jnp.all(cnt == jnp.minimum(rowg + 1, _K))

    def fast(_):
        def body(ki, carry):
            g, z = carry
            sb = s_ref[:, pl.ds(ki * _CBLK, _CBLK)]
            pb = jnp.where(sb >= t, jnp.exp(sb - m0), 0.0)
            z = z + jnp.sum(pb, axis=1, keepdims=True)
            vb = vp_ref[0, pl.ds(ki * _CBLK, _CBLK), :]
            g = g + lax.dot_general(
                pb.astype(jnp.bfloat16), vb, (((1,), (0,)), ((), ())),
                preferred_element_type=f32)
            return (g, z)
        return lax.fori_loop(
            0, qi + 1, body,
            (jnp.zeros((QB, D), f32), jnp.zeros((QB, 1), f32)))

    def slow(_):
        col = lax.broadcasted_iota(jnp.int32, (QB, N), 1)
        work = s
        for _ in range(_K):
            m = jnp.max(work, axis=1, keepdims=True)
            am = jnp.min(jnp.where(work == m, col, N), axis=1, keepdims=True)
            work = jnp.where(col == am, neg, work)
        mask = work != s
        p = jnp.where(mask, jnp.exp(s - m0), 0.0)
        z = jnp.sum(p, axis=1, keepdims=True)
        g = lax.dot_general(
            p.astype(jnp.bfloat16), vp_ref[0], (((1,), (0,)), ((), ())),
            preferred_element_type=f32)
        return (g, z)

    g, z = lax.cond(ok, fast, slow, 0)
    o_ref[0] = g / z + loc_ref[0]


def kernel(mu, Wq, bq, Wk, bk, Wv, bv, Wm1, bm1, Wm2, bm2, Wo, bo):
    B, N, D = mu.shape
    f32 = jnp.float32
    bf16 = jnp.bfloat16

    wvpT, wlT, bvp, ball = pl.pallas_call(
        _fold_kernel,
        out_shape=[
            jax.ShapeDtypeStruct((D, D), bf16),
            jax.ShapeDtypeStruct((D, D), bf16),
            jax.ShapeDtypeStruct((1, D), f32),
            jax.ShapeDtypeStruct((1, D), f32),
        ],
    )(Wv, Wm2, Wo, bv[None, :], bm2[None, :], bo[None, :])

    nblk = N // _QBLK
    kk, vp, loc = pl.pallas_call(
        _proj_kernel,
        grid=(B, nblk),
        in_specs=[
            pl.BlockSpec((1, _QBLK, D), lambda b, i: (b, i, 0)),
            pl.BlockSpec((D, D), lambda b, i: (0, 0)),
            pl.BlockSpec((D, D), lambda b, i: (0, 0)),
            pl.BlockSpec((D, 2 * D), lambda b, i: (0, 0)),
            pl.BlockSpec((D, D), lambda b, i: (0, 0)),
            pl.BlockSpec((1, D), lambda b, i: (0, 0)),
            pl.BlockSpec((1, D), lambda b, i: (0, 0)),
            pl.BlockSpec((1, D), lambda b, i: (0, 0)),
            pl.BlockSpec((1, D), lambda b, i: (0, 0)),
        ],
        out_specs=[
            pl.BlockSpec((1, _QBLK, D), lambda b, i: (b, i, 0)),
            pl.BlockSpec((1, _QBLK, D), lambda b, i: (b, i, 0)),
            pl.BlockSpec((1, _QBLK, D), lambda b, i: (b, i, 0)),
        ],
        out_shape=[
            jax.ShapeDtypeStruct((B, N, D), f32),
            jax.ShapeDtypeStruct((B, N, D), bf16),
            jax.ShapeDtypeStruct((B, N, D), f32),
        ],
        scratch_shapes=[pltpu.VMEM((8, D), f32)],
        compiler_params=pltpu.CompilerParams(
            dimension_semantics=("parallel", "arbitrary")),
    )(mu, Wk, wvpT, Wm1.astype(bf16), wlT,
      bk[None, :], bvp, bm1[None, :], ball)

    out = pl.pallas_call(
        _attn_kernel,
        grid=(B, nblk),
        in_specs=[
            pl.BlockSpec((1, _QBLK, D), lambda b, i: (b, i, 0)),
            pl.BlockSpec((D, D), lambda b, i: (0, 0)),
            pl.BlockSpec((1, D), lambda b, i: (0, 0)),
            pl.BlockSpec((1, N, D), lambda b, i: (b, 0, 0)),
            pl.BlockSpec((1, N, D), lambda b, i: (b, 0, 0)),
            pl.BlockSpec((1, _QBLK, D), lambda b, i: (b, i, 0)),
        ],
        out_specs=pl.BlockSpec((1, _QBLK, D), lambda b, i: (b, i, 0)),
        out_shape=jax.ShapeDtypeStruct((B, N, D), f32),
        scratch_shapes=[pltpu.VMEM((_QBLK, N), f32)],
        compiler_params=pltpu.CompilerParams(
            dimension_semantics=("parallel", "parallel")),
    )(mu, Wq, bq[None, :], kk, vp, loc)
    return out


# speedup vs baseline: 1.0948x; 1.0750x over previous
import functools
import math

import jax
import jax.numpy as jnp
from jax import lax
from jax.experimental import pallas as pl
from jax.experimental.pallas import tpu as pltpu

_WINDOW = 4
_K = 8
_QBLK = 256
_CBLK = 256


def _fold_kernel(wv_ref, wm2_ref, wo_ref, bv_ref, bm2_ref, bo_ref,
                 wvpT_ref, wlT_ref, bvp_ref, ball_ref):
    D = wv_ref.shape[0]
    wo = wo_ref[...]
    wol = wo[:, :D]
    wog = wo[:, D:]
    wv = wv_ref[...]
    wm2 = wm2_ref[...]
    wvpT_ref[...] = lax.dot_general(
        wv, wog, (((0,), (1,)), ((), ())),
        preferred_element_type=jnp.float32).astype(jnp.bfloat16)
    wlT_ref[...] = lax.dot_general(
        wm2, wol, (((0,), (1,)), ((), ())),
        preferred_element_type=jnp.float32).astype(jnp.bfloat16)
    zD = jnp.zeros((1, D), jnp.float32)
    lhs = jnp.concatenate([
        jnp.concatenate([bm2_ref[...], zD], axis=1),
        jnp.concatenate([zD, bv_ref[...]], axis=1),
    ], axis=0)
    y = lax.dot_general(lhs, wo, (((1,), (1,)), ((), ())),
                        preferred_element_type=jnp.float32)
    ball_ref[...] = bo_ref[...] + y[0:1]
    bvp_ref[...] = y[1:2]


def _mega_kernel(x_ref, wq_ref, bq_ref, wk_ref, bk_ref, wvpT_ref, bvp_ref,
                 wm1_ref, bm1_ref, wlT_ref, ball_ref, o_ref,
                 k_sc, vp_sc, s_ref, tail_ref):
    qi = pl.program_id(1)
    QB = x_ref.shape[1]
    D = x_ref.shape[2]
    N = k_sc.shape[0]
    nch = N // _CBLK
    scale = 1.0 / math.sqrt(D)
    neg = jnp.float32(-jnp.inf)
    f32 = jnp.float32
    bf16 = jnp.bfloat16
    x = x_ref[0]

    k_sc[pl.ds(qi * QB, QB), :] = lax.dot_general(
        x, wk_ref[...], (((1,), (1,)), ((), ())),
        preferred_element_type=f32) + bk_ref[...]
    xb = x.astype(bf16)
    vp = lax.dot_general(
        xb, wvpT_ref[...], (((1,), (0,)), ((), ())),
        preferred_element_type=f32) + bvp_ref[...]
    vp_sc[pl.ds(qi * QB, QB), :] = vp.astype(bf16)

    wm1 = wm1_ref[...]
    a = (lax.dot_general(
        xb, wm1[:, :D], (((1,), (1,)), ((), ())),
        preferred_element_type=f32) + bm1_ref[...]).astype(bf16)
    bn = lax.dot_general(
        xb, wm1[:, D:], (((1,), (1,)), ((), ())),
        preferred_element_type=f32).astype(bf16)
    prev = jnp.where(qi == 0, 0.0, tail_ref[...]).astype(bf16)
    acc = jnp.zeros(a.shape, f32)
    for w in range(1, _WINDOW + 1):
        sh = jnp.concatenate([prev[8 - w:, :], bn[:QB - w, :]], axis=0)
        z = a + sh
        acc = acc + (z * jax.nn.sigmoid(z)).astype(f32)
    tail_ref[...] = bn[QB - 8:, :].astype(f32)
    pre = (acc * (1.0 / _WINDOW)).astype(bf16)
    loc = lax.dot_general(
        pre, wlT_ref[...], (((1,), (0,)), ((), ())),
        preferred_element_type=f32) + ball_ref[...]

    q = lax.dot_general(
        x, wq_ref[...], (((1,), (1,)), ((), ())),
        preferred_element_type=f32) + bq_ref[...]
    rloc = lax.broadcasted_iota(jnp.int32, (QB, _CBLK), 0)
    cloc = lax.broadcasted_iota(jnp.int32, (QB, _CBLK), 1)

    def scores_body(ki, _):
        kblk = k_sc[pl.ds(ki * _CBLK, _CBLK), :]
        sb = lax.dot_general(
            q, kblk, (((1,), (1,)), ((), ())),
            preferred_element_type=f32) * scale
        sb = jnp.where(ki * _CBLK + cloc <= qi * QB + rloc, sb, neg)
        s_ref[:, pl.ds(ki * _CBLK, _CBLK)] = sb
        return 0

    def blank_body(ki, _):
        s_ref[:, pl.ds(ki * _CBLK, _CBLK)] = jnp.full((QB, _CBLK), neg, f32)
        return 0

    lax.fori_loop(0, qi + 1, scores_body, 0)
    lax.fori_loop(qi + 1, nch, blank_body, 0)

    s = s_ref[...]
    m0 = jnp.max(s, axis=1, keepdims=True)
    mprev = m0
    for _ in range(_K - 1):
        mprev = jnp.max(jnp.where(s < mprev, s, neg), axis=1, keepdims=True)
    t = mprev
    cnt = jnp.sum(((s >= t) & (s > neg)).astype(jnp.int32), axis=1,
                  keepdims=True)
    rowg = qi * QB + lax.broadcasted_iota(jnp.int32, (QB, 1), 0)
    ok = jnp.all(cnt == jnp.minimum(rowg + 1, _K))

    def fast(_):
        def body(ki, carry):
            g, z = carry
            sb = s_ref[:, pl.ds(ki * _CBLK, _CBLK)]
            pb = jnp.where(sb >= t, jnp.exp(sb - m0), 0.0)
            z = z + jnp.sum(pb, axis=1, keepdims=True)
            vb = vp_sc[pl.ds(ki * _CBLK, _CBLK), :]
            g = g + lax.dot_general(
                pb.astype(bf16), vb, (((1,), (0,)), ((), ())),
                preferred_element_type=f32)
            return (g, z)
        return lax.fori_loop(
            0, qi + 1, body,
            (jnp.zeros((QB, D), f32), jnp.zeros((QB, 1), f32)))

    def slow(_):
        col = lax.broadcasted_iota(jnp.int32, (QB, N), 1)
        work = s
        for _ in range(_K):
            m = jnp.max(work, axis=1, keepdims=True)
            am = jnp.min(jnp.where(work == m, col, N), axis=1, keepdims=True)
            work = jnp.where(col == am, neg, work)
        mask = work != s
        p = jnp.where(mask, jnp.exp(s - m0), 0.0)
        z = jnp.sum(p, axis=1, keepdims=True)
        g = lax.dot_general(
            p.astype(bf16), vp_sc[...], (((1,), (0,)), ((), ())),
            preferred_element_type=f32)
        return (g, z)

    g, z = lax.cond(ok, fast, slow, 0)
    o_ref[0] = g / z + loc


def kernel(mu, Wq, bq, Wk, bk, Wv, bv, Wm1, bm1, Wm2, bm2, Wo, bo):
    B, N, D = mu.shape
    f32 = jnp.float32
    bf16 = jnp.bfloat16

    wvpT, wlT, bvp, ball = pl.pallas_call(
        _fold_kernel,
        out_shape=[
            jax.ShapeDtypeStruct((D, D), bf16),
            jax.ShapeDtypeStruct((D, D), bf16),
            jax.ShapeDtypeStruct((1, D), f32),
            jax.ShapeDtypeStruct((1, D), f32),
        ],
    )(Wv, Wm2, Wo, bv[None, :], bm2[None, :], bo[None, :])

    nblk = N // _QBLK
    whole = lambda shape: pl.BlockSpec(shape, lambda b, i: tuple(0 for _ in shape))
    out = pl.pallas_call(
        _mega_kernel,
        grid=(B, nblk),
        in_specs=[
            pl.BlockSpec((1, _QBLK, D), lambda b, i: (b, i, 0)),
            whole((D, D)),
            whole((1, D)),
            whole((D, D)),
            whole((1, D)),
            whole((D, D)),
            whole((1, D)),
            whole((D, 2 * D)),
            whole((1, D)),
            whole((D, D)),
            whole((1, D)),
        ],
        out_specs=pl.BlockSpec((1, _QBLK, D), lambda b, i: (b, i, 0)),
        out_shape=jax.ShapeDtypeStruct((B, N, D), f32),
        scratch_shapes=[
            pltpu.VMEM((N, D), f32),
            pltpu.VMEM((N, D), bf16),
            pltpu.VMEM((_QBLK, N), f32),
            pltpu.VMEM((8, D), f32),
        ],
    )(mu, Wq, bq[None, :], Wk, bk[None, :], wvpT, bvp,
      Wm1.astype(bf16), bm1[None, :], wlT, ball)
    return out


# speedup vs baseline: 1.1830x; 1.0806x over previous
import functools
import math

import jax
import jax.numpy as jnp
from jax import lax
from jax.experimental import pallas as pl
from jax.experimental.pallas import tpu as pltpu

_WINDOW = 4
_K = 8
_QBLK = 512
_CBLK = 256


def _fold_kernel(wv_ref, wm2_ref, wo_ref, bv_ref, bm2_ref, bo_ref,
                 wvpT_ref, wlT_ref, bvp_ref, ball_ref):
    D = wv_ref.shape[0]
    wo = wo_ref[...]
    wol = wo[:, :D]
    wog = wo[:, D:]
    wv = wv_ref[...]
    wm2 = wm2_ref[...]
    wvpT_ref[...] = lax.dot_general(
        wv, wog, (((0,), (1,)), ((), ())),
        preferred_element_type=jnp.float32).astype(jnp.bfloat16)
    wlT_ref[...] = lax.dot_general(
        wm2, wol, (((0,), (1,)), ((), ())),
        preferred_element_type=jnp.float32).astype(jnp.bfloat16)
    zD = jnp.zeros((1, D), jnp.float32)
    lhs = jnp.concatenate([
        jnp.concatenate([bm2_ref[...], zD], axis=1),
        jnp.concatenate([zD, bv_ref[...]], axis=1),
    ], axis=0)
    y = lax.dot_general(lhs, wo, (((1,), (1,)), ((), ())),
                        preferred_element_type=jnp.float32)
    ball_ref[...] = bo_ref[...] + y[0:1]
    bvp_ref[...] = y[1:2]


def _mega_kernel(x_ref, wq_ref, bq_ref, wk_ref, bk_ref, wvpT_ref, bvp_ref,
                 wm1_ref, bm1_ref, wlT_ref, ball_ref, o_ref,
                 k_sc, vp_sc, s_ref, tail_ref):
    qi = pl.program_id(1)
    QB = x_ref.shape[1]
    D = x_ref.shape[2]
    N = k_sc.shape[0]
    nch = N // _CBLK
    scale = 1.0 / math.sqrt(D)
    neg = jnp.float32(-jnp.inf)
    f32 = jnp.float32
    bf16 = jnp.bfloat16
    x = x_ref[0]

    k_sc[pl.ds(qi * QB, QB), :] = lax.dot_general(
        x, wk_ref[...], (((1,), (1,)), ((), ())),
        preferred_element_type=f32) + bk_ref[...]
    xb = x.astype(bf16)
    vp = lax.dot_general(
        xb, wvpT_ref[...], (((1,), (0,)), ((), ())),
        preferred_element_type=f32) + bvp_ref[...]
    vp_sc[pl.ds(qi * QB, QB), :] = vp.astype(bf16)

    wm1 = wm1_ref[...]
    a = (lax.dot_general(
        xb, wm1[:, :D], (((1,), (1,)), ((), ())),
        preferred_element_type=f32) + bm1_ref[...]).astype(bf16)
    bn = lax.dot_general(
        xb, wm1[:, D:], (((1,), (1,)), ((), ())),
        preferred_element_type=f32).astype(bf16)
    prev = jnp.where(qi == 0, 0.0, tail_ref[...]).astype(bf16)
    acc = jnp.zeros(a.shape, f32)
    for w in range(1, _WINDOW + 1):
        sh = jnp.concatenate([prev[8 - w:, :], bn[:QB - w, :]], axis=0)
        z = a + sh
        acc = acc + (z * jax.nn.sigmoid(z)).astype(f32)
    tail_ref[...] = bn[QB - 8:, :].astype(f32)
    pre = (acc * (1.0 / _WINDOW)).astype(bf16)
    loc = lax.dot_general(
        pre, wlT_ref[...], (((1,), (0,)), ((), ())),
        preferred_element_type=f32) + ball_ref[...]

    q = lax.dot_general(
        x, wq_ref[...], (((1,), (1,)), ((), ())),
        preferred_element_type=f32) + bq_ref[...]
    rloc = lax.broadcasted_iota(jnp.int32, (QB, _CBLK), 0)
    cloc = lax.broadcasted_iota(jnp.int32, (QB, _CBLK), 1)

    def scores_body(ki, _):
        kblk = k_sc[pl.ds(ki * _CBLK, _CBLK), :]
        sb = lax.dot_general(
            q, kblk, (((1,), (1,)), ((), ())),
            preferred_element_type=f32) * scale
        sb = jnp.where(ki * _CBLK + cloc <= qi * QB + rloc, sb, neg)
        s_ref[:, pl.ds(ki * _CBLK, _CBLK)] = sb
        return 0

    def blank_body(ki, _):
        s_ref[:, pl.ds(ki * _CBLK, _CBLK)] = jnp.full((QB, _CBLK), neg, f32)
        return 0

    nval = (qi + 1) * (QB // _CBLK)
    lax.fori_loop(0, nval, scores_body, 0)
    lax.fori_loop(nval, nch, blank_body, 0)

    s = s_ref[...]
    m0 = jnp.max(s, axis=1, keepdims=True)
    mprev = m0
    for _ in range(_K - 1):
        mprev = jnp.max(jnp.where(s < mprev, s, neg), axis=1, keepdims=True)
    t = mprev
    cnt = jnp.sum(((s >= t) & (s > neg)).astype(jnp.int32), axis=1,
                  keepdims=True)
    rowg = qi * QB + lax.broadcasted_iota(jnp.int32, (QB, 1), 0)
    ok = jnp.all(cnt == jnp.minimum(rowg + 1, _K))

    def fast(_):
        def body(ki, carry):
            g, z = carry
            sb = s_ref[:, pl.ds(ki * _CBLK, _CBLK)]
            pb = jnp.where(sb >= t, jnp.exp(sb - m0), 0.0)
            z = z + jnp.sum(pb, axis=1, keepdims=True)
            vb = vp_sc[pl.ds(ki * _CBLK, _CBLK), :]
            g = g + lax.dot_general(
                pb.astype(bf16), vb, (((1,), (0,)), ((), ())),
                preferred_element_type=f32)
            return (g, z)
        return lax.fori_loop(
            0, nval, body,
            (jnp.zeros((QB, D), f32), jnp.zeros((QB, 1), f32)))

    def slow(_):
        col = lax.broadcasted_iota(jnp.int32, (QB, N), 1)
        work = s
        for _ in range(_K):
            m = jnp.max(work, axis=1, keepdims=True)
            am = jnp.min(jnp.where(work == m, col, N), axis=1, keepdims=True)
            work = jnp.where(col == am, neg, work)
        mask = work != s
        p = jnp.where(mask, jnp.exp(s - m0), 0.0)
        z = jnp.sum(p, axis=1, keepdims=True)
        g = lax.dot_general(
            p.astype(bf16), vp_sc[...], (((1,), (0,)), ((), ())),
            preferred_element_type=f32)
        return (g, z)

    g, z = lax.cond(ok, fast, slow, 0)
    o_ref[0] = g / z + loc


def kernel(mu, Wq, bq, Wk, bk, Wv, bv, Wm1, bm1, Wm2, bm2, Wo, bo):
    B, N, D = mu.shape
    f32 = jnp.float32
    bf16 = jnp.bfloat16

    wvpT, wlT, bvp, ball = pl.pallas_call(
        _fold_kernel,
        out_shape=[
            jax.ShapeDtypeStruct((D, D), bf16),
            jax.ShapeDtypeStruct((D, D), bf16),
            jax.ShapeDtypeStruct((1, D), f32),
            jax.ShapeDtypeStruct((1, D), f32),
        ],
    )(Wv, Wm2, Wo, bv[None, :], bm2[None, :], bo[None, :])

    nblk = N // _QBLK
    whole = lambda shape: pl.BlockSpec(shape, lambda b, i: tuple(0 for _ in shape))
    out = pl.pallas_call(
        _mega_kernel,
        grid=(B, nblk),
        in_specs=[
            pl.BlockSpec((1, _QBLK, D), lambda b, i: (b, i, 0)),
            whole((D, D)),
            whole((1, D)),
            whole((D, D)),
            whole((1, D)),
            whole((D, D)),
            whole((1, D)),
            whole((D, 2 * D)),
            whole((1, D)),
            whole((D, D)),
            whole((1, D)),
        ],
        out_specs=pl.BlockSpec((1, _QBLK, D), lambda b, i: (b, i, 0)),
        out_shape=jax.ShapeDtypeStruct((B, N, D), f32),
        scratch_shapes=[
            pltpu.VMEM((N, D), f32),
            pltpu.VMEM((N, D), bf16),
            pltpu.VMEM((_QBLK, N), f32),
            pltpu.VMEM((8, D), f32),
        ],
    )(mu, Wq, bq[None, :], Wk, bk[None, :], wvpT, bvp,
      Wm1.astype(bf16), bm1[None, :], wlT, ball)
    return out
